# baseline (speedup 1.0000x reference)
import functools

import jax
import jax.numpy as jnp
from jax import lax
from jax.experimental import pallas as pl
from jax.experimental.pallas import tpu as pltpu
from jax.experimental.pallas import tpu_sc as plsc

N = 10000
E = 320000
D = 128
NEG = 0.2
EPS = 1e-16

NPAD = 10240
NBLK = 164
NROWS = 16 * NBLK
DH = D // 2
E_TOT = E + N
ROWS_PER_TILE = NBLK
NSLICE = NPAD // 16
NBUF = 4
CB = 4
NCH = NBLK // CB
GA = 2


def _tc_first(x_ref, w_ref, asrc_ref, adst_ref, h_ref, a_ref, d_ref):
    h = jnp.dot(x_ref[...], w_ref[...], preferred_element_type=jnp.float32)
    h_ref[0] = h[:, 0:DH]
    h_ref[1] = h[:, DH:D]
    a_ref[0:N, :] = jnp.sum(h * asrc_ref[...], axis=1, keepdims=True)
    a_ref[N:NPAD, :] = jnp.zeros((NPAD - N, 1), jnp.float32)
    d_ref[0:N, :] = jnp.sum(h * adst_ref[...], axis=1, keepdims=True)
    d_ref[N:NPAD, :] = jnp.zeros((NPAD - N, 1), jnp.float32)


def _combine(p_ref, dn_ref):
    mask = jnp.concatenate([jnp.ones((16, D), jnp.float32),
                            jnp.zeros((16, D), jnp.float32)], axis=0)
    dsum = lax.dot_general(dn_ref[...], mask,
                           (((0,), (0,)), ((), ())),
                           preferred_element_type=jnp.float32)
    tot = jnp.concatenate([p_ref[0], p_ref[1]], axis=1)
    return tot[0:N, :] / (dsum[0:N, :] + EPS)


def _tc_mid(p_ref, dn_ref, b_ref, w_ref, asrc_ref, adst_ref,
            h_ref, a_ref, d_ref):
    x1 = _combine(p_ref, dn_ref) + b_ref[...]
    x1r = jnp.maximum(x1, 0.0)
    h = jnp.dot(x1r, w_ref[...], preferred_element_type=jnp.float32)
    h_ref[0] = h[:, 0:DH]
    h_ref[1] = h[:, DH:D]
    a_ref[0:N, :] = jnp.sum(h * asrc_ref[...], axis=1, keepdims=True)
    a_ref[N:NPAD, :] = jnp.zeros((NPAD - N, 1), jnp.float32)
    d_ref[0:N, :] = jnp.sum(h * adst_ref[...], axis=1, keepdims=True)
    d_ref[N:NPAD, :] = jnp.zeros((NPAD - N, 1), jnp.float32)


def _tc_final(p_ref, dn_ref, b_ref, out_ref):
    out_ref[...] = _combine(p_ref, dn_ref) + b_ref[...]


def _lrelu(v):
    return jnp.where(v >= 0.0, v, NEG * v)


def _sc_body(h_hbm, as_hbm, ad_hbm, src_hbm, dst_hbm, out_hbm, dn_hbm,
             as_v, ad_v, denom_v, src_i, dst_i, g_blk, rows_v, acc_sh,
             sem_g0, sem_g1, sem_g2, sem_g3,
             sem_s0, sem_s1, sem_s2, sem_s3, sem_i0, sem_i1):
    c = lax.axis_index("c")
    s = lax.axis_index("s")
    hh = h_hbm.at[c]

    pltpu.sync_copy(as_hbm, as_v)
    pltpu.sync_copy(ad_hbm, ad_v)

    def _mx(i, m):
        return jnp.maximum(m, as_v[pl.ds(i * 16, 16)])
    mvec = lax.fori_loop(0, NPAD // 16, _mx,
                         jnp.full((16,), -1e30, jnp.float32))
    a_max = mvec[0]
    for l in range(1, 16):
        a_max = jnp.maximum(a_max, mvec[l])

    zero16 = jnp.zeros((16,), jnp.float32)

    def _z1(i, _):
        denom_v[pl.ds(i * 16, 16)] = zero16
        return 0
    lax.fori_loop(0, NPAD // 16, _z1, 0)

    def _z2(i, _):
        for v in range(DH // 16):
            rows_v[0, i, pl.ds(v * 16, 16)] = zero16
        return 0
    lax.fori_loop(0, 128, _z2, 0)

    base = s * NSLICE
    for k in range(NSLICE // 128):
        pltpu.sync_copy(rows_v.at[0],
                        acc_sh.at[pl.ds(base + 128 * k, 128)])
    plsc.subcore_barrier()

    sem_g = [sem_g0, sem_g1, sem_g2, sem_g3]
    sem_s = [sem_s0, sem_s1, sem_s2, sem_s3]
    sem_i = [sem_i0, sem_i1]

    sh = src_hbm.at[s]
    dh = dst_hbm.at[s]

    def _load_idx(t, p, sync):
        if sync:
            pltpu.sync_copy(sh.at[t], src_i.at[p])
            pltpu.sync_copy(dh.at[t], dst_i.at[p])
        else:
            pltpu.async_copy(sh.at[t], src_i.at[p], sem_i[p])
            pltpu.async_copy(dh.at[t], dst_i.at[p], sem_i[p])

    def _wait_idx(p):
        pltpu.make_async_copy(sh.at[0], src_i.at[p], sem_i[p]).wait()
        pltpu.make_async_copy(dh.at[0], dst_i.at[p], sem_i[p]).wait()

    def _step(k, u, p, pn):
        b = u
        bg = (b + GA) % NBUF

        @pl.when(k >= NBUF - GA)
        def _():
            pltpu.make_async_copy(rows_v.at[bg], acc_sh.at[dst_i.at[p, u]],
                                  sem_s[bg]).wait()

        nu = u + GA
        nsrc = src_i.at[p, nu] if nu < CB else src_i.at[pn, nu - CB]

        @pl.when(k + GA < NBLK)
        def _():
            pltpu.async_copy(hh.at[nsrc], rows_v.at[bg], sem_g[bg])

        for kk in range(8):
            sl = pl.ds(kk * 16, 16)
            s16 = src_i[p, u, sl]
            d16 = dst_i[p, u, sl]
            asv = plsc.load_gather(as_v, [s16])
            adv = plsc.load_gather(ad_v, [d16])
            g = jnp.exp(_lrelu(asv + adv) - _lrelu(a_max + adv))
            g_blk[b, sl] = g
            plsc.addupdate_scatter(denom_v, [d16], g)

        pltpu.make_async_copy(hh.at[src_i.at[p, u]], rows_v.at[b],
                              sem_g[b]).wait()

        ib = jnp.full((16,), b, jnp.int32)

        def _scale(k2, _):
            e0 = k2 * 16
            for l in range(16):
                e = e0 + l
                av = plsc.load_gather(g_blk, [ib, jnp.zeros((16,), jnp.int32) + e])
                for v in range(DH // 16):
                    vsl = pl.ds(v * 16, 16)
                    rows_v[b, e, vsl] = rows_v[b, e, vsl] * av
            return 0
        lax.fori_loop(0, 8, _scale, 0)

        pltpu.async_copy(rows_v.at[b], acc_sh.at[dst_i.at[p, u]], sem_s[b],
                         add=True)

    def _chunk(t, p, last):
        pn = 1 - p
        for u in range(CB - GA):
            _step(CB * t + u, u, p, pn)

        @pl.when(jnp.logical_not(last))
        def _():
            _wait_idx(pn)
        for u in range(CB - GA, CB):
            _step(CB * t + u, u, p, pn)
        @pl.when(t + 2 < NCH)
        def _():
            _load_idx(t + 2, p, sync=False)

    _load_idx(0, 0, sync=True)
    _load_idx(1, 1, sync=False)
    for k in range(GA):
        pltpu.async_copy(hh.at[src_i.at[0, k]], rows_v.at[k], sem_g[k])
    _chunk(0, 0, jnp.bool_(False))

    def _pair(i, _):
        _chunk(2 * i + 1, 1, jnp.bool_(False))
        _chunk(2 * i + 2, 0, jnp.bool_(False))
        return 0
    lax.fori_loop(0, (NCH - 3) // 2, _pair, 0)
    _chunk(NCH - 2, 1, jnp.bool_(False))
    _chunk(NCH - 1, 0, jnp.bool_(True))

    for b in range(GA, NBUF):
        pltpu.make_async_copy(rows_v.at[b], acc_sh.at[dst_i.at[0, 0]],
                              sem_s[b]).wait()

    plsc.subcore_barrier()

    for k in range(NSLICE // 128):
        pltpu.sync_copy(acc_sh.at[pl.ds(base + 128 * k, 128)],
                        out_hbm.at[c, pl.ds(base + 128 * k, 128)])
    pltpu.sync_copy(denom_v, dn_hbm.at[c * 16 + s])


def _sc_layer(h, asv, adv, srcm, dstm):
    mesh = plsc.VectorSubcoreMesh(core_axis_name="c", subcore_axis_name="s")
    fn = pl.kernel(
        _sc_body,
        out_type=[jax.ShapeDtypeStruct((2, NPAD, DH), jnp.float32),
                  jax.ShapeDtypeStruct((32, NPAD), jnp.float32)],
        mesh=mesh,
        compiler_params=pltpu.CompilerParams(needs_layout_passes=False,
                                             use_tc_tiling_on_sc=False),
        scratch_types=[
            pltpu.VMEM((NPAD,), jnp.float32),
            pltpu.VMEM((NPAD,), jnp.float32),
            pltpu.VMEM((NPAD,), jnp.float32),
            pltpu.VMEM((2, CB, 128), jnp.int32),
            pltpu.VMEM((2, CB, 128), jnp.int32),
            pltpu.VMEM((NBUF, 128), jnp.float32),
            pltpu.VMEM((NBUF, 128, DH), jnp.float32),
            pltpu.VMEM_SHARED((NPAD, DH), jnp.float32),
        ] + [pltpu.SemaphoreType.DMA] * 10,
    )
    return fn(h, asv, adv, srcm, dstm)


def kernel(x, edge_index, W1, a_src1, a_dst1, b1, W2, a_src2, a_dst2, b2):
    src = edge_index[0]
    dst = edge_index[1]
    loops = jnp.arange(N, dtype=jnp.int32)
    npad_e = NROWS * 128 - E_TOT
    srcm = jnp.concatenate(
        [src, loops,
         jnp.zeros((npad_e,), jnp.int32)]).reshape(16, NCH, CB, 128)
    dstm = jnp.concatenate(
        [dst, loops,
         jnp.full((npad_e,), N, jnp.int32)]).reshape(16, NCH, CB, 128)

    tc1 = pl.pallas_call(
        _tc_first,
        out_shape=(jax.ShapeDtypeStruct((2, N, DH), jnp.float32),
                   jax.ShapeDtypeStruct((NPAD, 1), jnp.float32),
                   jax.ShapeDtypeStruct((NPAD, 1), jnp.float32)),
    )
    h1, as1, ad1 = tc1(x, W1, a_src1.reshape(1, D), a_dst1.reshape(1, D))

    p1, dn1 = _sc_layer(h1, as1.reshape(NPAD), ad1.reshape(NPAD), srcm, dstm)

    tc2 = pl.pallas_call(
        _tc_mid,
        out_shape=(jax.ShapeDtypeStruct((2, N, DH), jnp.float32),
                   jax.ShapeDtypeStruct((NPAD, 1), jnp.float32),
                   jax.ShapeDtypeStruct((NPAD, 1), jnp.float32)),
    )
    h2, as2, ad2 = tc2(p1, dn1, b1.reshape(1, D), W2,
                       a_src2.reshape(1, D), a_dst2.reshape(1, D))

    p2, dn2 = _sc_layer(h2, as2.reshape(NPAD), ad2.reshape(NPAD), srcm, dstm)

    tc3 = pl.pallas_call(
        _tc_final,
        out_shape=jax.ShapeDtypeStruct((N, D), jnp.float32),
    )
    return tc3(p2, dn2, b2.reshape(1, D))


# speedup vs baseline: 1.2745x; 1.2745x over previous
import functools

import jax
import jax.numpy as jnp
from jax import lax
from jax.experimental import pallas as pl
from jax.experimental.pallas import tpu as pltpu
from jax.experimental.pallas import tpu_sc as plsc

N = 10000
E = 320000
D = 128
NEG = 0.2
EPS = 1e-16

NPAD = 10240
NBLK = 164
NROWS = 16 * NBLK
DH = D // 2
E_TOT = E + N
ROWS_PER_TILE = NBLK
NSLICE = NPAD // 16
NBUF = 4
CB = 4
NCH = NBLK // CB
GA = 2


def _tc_first(x_ref, w_ref, asrc_ref, adst_ref, h_ref, a_ref, d_ref):
    h = jnp.dot(x_ref[...], w_ref[...], preferred_element_type=jnp.float32)
    h_ref[0] = h[:, 0:DH]
    h_ref[1] = h[:, DH:D]
    a_ref[0:N, :] = jnp.sum(h * asrc_ref[...], axis=1, keepdims=True)
    a_ref[N:NPAD, :] = jnp.zeros((NPAD - N, 1), jnp.float32)
    d_ref[0:N, :] = jnp.sum(h * adst_ref[...], axis=1, keepdims=True)
    d_ref[N:NPAD, :] = jnp.zeros((NPAD - N, 1), jnp.float32)


def _combine(p_ref, dn_ref):
    mask = jnp.concatenate([jnp.ones((16, D), jnp.float32),
                            jnp.zeros((16, D), jnp.float32)], axis=0)
    dsum = lax.dot_general(dn_ref[...], mask,
                           (((0,), (0,)), ((), ())),
                           preferred_element_type=jnp.float32)
    tot = jnp.concatenate([p_ref[0], p_ref[1]], axis=1)
    return tot[0:N, :] / (dsum[0:N, :] + EPS)


def _tc_mid(p_ref, dn_ref, b_ref, w_ref, asrc_ref, adst_ref,
            h_ref, a_ref, d_ref):
    x1 = _combine(p_ref, dn_ref) + b_ref[...]
    x1r = jnp.maximum(x1, 0.0)
    h = jnp.dot(x1r, w_ref[...], preferred_element_type=jnp.float32)
    h_ref[0] = h[:, 0:DH]
    h_ref[1] = h[:, DH:D]
    a_ref[0:N, :] = jnp.sum(h * asrc_ref[...], axis=1, keepdims=True)
    a_ref[N:NPAD, :] = jnp.zeros((NPAD - N, 1), jnp.float32)
    d_ref[0:N, :] = jnp.sum(h * adst_ref[...], axis=1, keepdims=True)
    d_ref[N:NPAD, :] = jnp.zeros((NPAD - N, 1), jnp.float32)


def _tc_final(p_ref, dn_ref, b_ref, out_ref):
    out_ref[...] = _combine(p_ref, dn_ref) + b_ref[...]


def _lrelu(v):
    return jnp.where(v >= 0.0, v, NEG * v)


def _sc_body(h_hbm, as_hbm, ad_hbm, src_hbm, dst_hbm, out_hbm, dn_hbm,
             as_v, ad_v, denom_v, src_i, dst_i, g_blk, rows_v, acc_sh,
             sem_g0, sem_g1, sem_g2, sem_g3,
             sem_s0, sem_s1, sem_s2, sem_s3, sem_i0, sem_i1):
    c = lax.axis_index("c")
    s = lax.axis_index("s")
    hh = h_hbm.at[c]

    pltpu.sync_copy(as_hbm, as_v)
    pltpu.sync_copy(ad_hbm, ad_v)

    def _mx(i, m):
        return jnp.maximum(m, as_v[pl.ds(i * 16, 16)])
    mvec = lax.fori_loop(0, NPAD // 16, _mx,
                         jnp.full((16,), -1e30, jnp.float32))
    a_max = mvec[0]
    for l in range(1, 16):
        a_max = jnp.maximum(a_max, mvec[l])

    zero16 = jnp.zeros((16,), jnp.float32)

    def _z1(i, _):
        denom_v[pl.ds(i * 16, 16)] = zero16
        return 0
    lax.fori_loop(0, NPAD // 16, _z1, 0)

    def _z2(i, _):
        for v in range(DH // 16):
            rows_v[0, i, pl.ds(v * 16, 16)] = zero16
        return 0
    lax.fori_loop(0, 128, _z2, 0)

    base = s * NSLICE
    for k in range(NSLICE // 128):
        pltpu.sync_copy(rows_v.at[0],
                        acc_sh.at[pl.ds(base + 128 * k, 128)])
    plsc.subcore_barrier()

    sem_g = [sem_g0, sem_g1, sem_g2, sem_g3]
    sem_s = [sem_s0, sem_s1, sem_s2, sem_s3]
    sem_i = [sem_i0, sem_i1]

    sh = src_hbm.at[s]
    dh = dst_hbm.at[s]

    def _load_idx(t, p, sync):
        if sync:
            pltpu.sync_copy(sh.at[t], src_i.at[p])
            pltpu.sync_copy(dh.at[t], dst_i.at[p])
        else:
            pltpu.async_copy(sh.at[t], src_i.at[p], sem_i[p])
            pltpu.async_copy(dh.at[t], dst_i.at[p], sem_i[p])

    def _wait_idx(p):
        pltpu.make_async_copy(sh.at[0], src_i.at[p], sem_i[p]).wait()
        pltpu.make_async_copy(dh.at[0], dst_i.at[p], sem_i[p]).wait()

    def _step(k, u, p, pn):
        b = u
        bg = (b + GA) % NBUF

        @pl.when(k >= NBUF - GA)
        def _():
            pltpu.make_async_copy(rows_v.at[bg], acc_sh.at[dst_i.at[p, u]],
                                  sem_s[bg]).wait()

        nu = u + GA
        nsrc = src_i.at[p, nu] if nu < CB else src_i.at[pn, nu - CB]

        @pl.when(k + GA < NBLK)
        def _():
            pltpu.async_copy(hh.at[nsrc], rows_v.at[bg], sem_g[bg])

        for kk in range(8):
            sl = pl.ds(kk * 16, 16)
            s16 = src_i[p, u, sl]
            d16 = dst_i[p, u, sl]
            asv = plsc.load_gather(as_v, [s16])
            adv = plsc.load_gather(ad_v, [d16])
            g = jnp.exp(_lrelu(asv + adv) - _lrelu(a_max + adv))
            g_blk[b, sl] = g
            plsc.addupdate_scatter(denom_v, [d16], g)

        pltpu.make_async_copy(hh.at[src_i.at[p, u]], rows_v.at[b],
                              sem_g[b]).wait()

        @functools.partial(plsc.parallel_loop, 0, 8, unroll=2)
        def _scale(k2):
            g16 = g_blk[b, pl.ds(k2 * 16, 16)]
            for l in range(16):
                a = g16[l]
                e = k2 * 16 + l
                for v in range(DH // 16):
                    vsl = pl.ds(v * 16, 16)
                    rows_v[b, e, vsl] = rows_v[b, e, vsl] * a

        pltpu.async_copy(rows_v.at[b], acc_sh.at[dst_i.at[p, u]], sem_s[b],
                         add=True)

    def _chunk(t, p, last):
        pn = 1 - p
        for u in range(CB - GA):
            _step(CB * t + u, u, p, pn)

        @pl.when(jnp.logical_not(last))
        def _():
            _wait_idx(pn)
        for u in range(CB - GA, CB):
            _step(CB * t + u, u, p, pn)
        @pl.when(t + 2 < NCH)
        def _():
            _load_idx(t + 2, p, sync=False)

    _load_idx(0, 0, sync=True)
    _load_idx(1, 1, sync=False)
    for k in range(GA):
        pltpu.async_copy(hh.at[src_i.at[0, k]], rows_v.at[k], sem_g[k])
    _chunk(0, 0, jnp.bool_(False))

    def _pair(i, _):
        _chunk(2 * i + 1, 1, jnp.bool_(False))
        _chunk(2 * i + 2, 0, jnp.bool_(False))
        return 0
    lax.fori_loop(0, (NCH - 3) // 2, _pair, 0)
    _chunk(NCH - 2, 1, jnp.bool_(False))
    _chunk(NCH - 1, 0, jnp.bool_(True))

    for b in range(GA, NBUF):
        pltpu.make_async_copy(rows_v.at[b], acc_sh.at[dst_i.at[0, 0]],
                              sem_s[b]).wait()

    plsc.subcore_barrier()

    for k in range(NSLICE // 128):
        pltpu.sync_copy(acc_sh.at[pl.ds(base + 128 * k, 128)],
                        out_hbm.at[c, pl.ds(base + 128 * k, 128)])
    pltpu.sync_copy(denom_v, dn_hbm.at[c * 16 + s])


def _sc_layer(h, asv, adv, srcm, dstm):
    mesh = plsc.VectorSubcoreMesh(core_axis_name="c", subcore_axis_name="s")
    fn = pl.kernel(
        _sc_body,
        out_type=[jax.ShapeDtypeStruct((2, NPAD, DH), jnp.float32),
                  jax.ShapeDtypeStruct((32, NPAD), jnp.float32)],
        mesh=mesh,
        compiler_params=pltpu.CompilerParams(needs_layout_passes=False,
                                             use_tc_tiling_on_sc=False),
        scratch_types=[
            pltpu.VMEM((NPAD,), jnp.float32),
            pltpu.VMEM((NPAD,), jnp.float32),
            pltpu.VMEM((NPAD,), jnp.float32),
            pltpu.VMEM((2, CB, 128), jnp.int32),
            pltpu.VMEM((2, CB, 128), jnp.int32),
            pltpu.VMEM((NBUF, 128), jnp.float32),
            pltpu.VMEM((NBUF, 128, DH), jnp.float32),
            pltpu.VMEM_SHARED((NPAD, DH), jnp.float32),
        ] + [pltpu.SemaphoreType.DMA] * 10,
    )
    return fn(h, asv, adv, srcm, dstm)


def kernel(x, edge_index, W1, a_src1, a_dst1, b1, W2, a_src2, a_dst2, b2):
    src = edge_index[0]
    dst = edge_index[1]
    loops = jnp.arange(N, dtype=jnp.int32)
    npad_e = NROWS * 128 - E_TOT
    srcm = jnp.concatenate(
        [src, loops,
         jnp.zeros((npad_e,), jnp.int32)]).reshape(16, NCH, CB, 128)
    dstm = jnp.concatenate(
        [dst, loops,
         jnp.full((npad_e,), N, jnp.int32)]).reshape(16, NCH, CB, 128)

    tc1 = pl.pallas_call(
        _tc_first,
        out_shape=(jax.ShapeDtypeStruct((2, N, DH), jnp.float32),
                   jax.ShapeDtypeStruct((NPAD, 1), jnp.float32),
                   jax.ShapeDtypeStruct((NPAD, 1), jnp.float32)),
    )
    h1, as1, ad1 = tc1(x, W1, a_src1.reshape(1, D), a_dst1.reshape(1, D))

    p1, dn1 = _sc_layer(h1, as1.reshape(NPAD), ad1.reshape(NPAD), srcm, dstm)

    tc2 = pl.pallas_call(
        _tc_mid,
        out_shape=(jax.ShapeDtypeStruct((2, N, DH), jnp.float32),
                   jax.ShapeDtypeStruct((NPAD, 1), jnp.float32),
                   jax.ShapeDtypeStruct((NPAD, 1), jnp.float32)),
    )
    h2, as2, ad2 = tc2(p1, dn1, b1.reshape(1, D), W2,
                       a_src2.reshape(1, D), a_dst2.reshape(1, D))

    p2, dn2 = _sc_layer(h2, as2.reshape(NPAD), ad2.reshape(NPAD), srcm, dstm)

    tc3 = pl.pallas_call(
        _tc_final,
        out_shape=jax.ShapeDtypeStruct((N, D), jnp.float32),
    )
    return tc3(p2, dn2, b2.reshape(1, D))
